# baseline (speedup 1.0000x reference)
import functools

import jax
import jax.numpy as jnp
from jax import lax
from jax.experimental import pallas as pl
from jax.experimental.pallas import tpu as pltpu
from jax.experimental.pallas import tpu_sc as plsc

_D = 128
_L = 50
_E = 8
_EPS = 1e-5
_CAP = 128
_GB = 128

_NC, _NS = 2, 16
_NW = _NC * _NS


def _sc_gather(table, idx, chunk):
    m, d = idx.shape[0], table.shape[1]
    per_w = m // _NW
    n_chunks = per_w // chunk
    mesh = plsc.VectorSubcoreMesh(
        core_axis_name="c", subcore_axis_name="s",
        num_cores=_NC, num_subcores=_NS)

    @functools.partial(
        pl.kernel, mesh=mesh,
        out_type=jax.ShapeDtypeStruct((m, d), jnp.float32),
        scratch_types=[
            pltpu.VMEM((chunk,), jnp.int32),
            pltpu.VMEM((chunk, d), jnp.float32),
            pltpu.SemaphoreType.DMA,
        ],
    )
    def k(table_hbm, idx_hbm, out_hbm, idx_v, rows_v, sem):
        wid = lax.axis_index("s") * _NC + lax.axis_index("c")
        base = wid * per_w

        def body(c, carry):
            off = base + c * chunk
            pltpu.sync_copy(idx_hbm.at[pl.ds(off, chunk)], idx_v)
            pltpu.async_copy(table_hbm.at[idx_v], rows_v, sem).wait()
            pltpu.sync_copy(rows_v, out_hbm.at[pl.ds(off, chunk)])
            return carry

        lax.fori_loop(0, n_chunks, body, 0)

    return k(table, idx)


def _shifted_conv(x_flat, n_tok, w_ref, b_ref, e_idx):
    cout = w_ref.shape[3]
    t0 = jnp.dot(x_flat, w_ref[e_idx, 0], preferred_element_type=jnp.float32)
    t1 = jnp.dot(x_flat, w_ref[e_idx, 1], preferred_element_type=jnp.float32)
    t2 = jnp.dot(x_flat, w_ref[e_idx, 2], preferred_element_type=jnp.float32)
    t0 = t0.reshape(n_tok, _L, cout)
    t1 = t1.reshape(n_tok, _L, cout)
    t2 = t2.reshape(n_tok, _L, cout)
    z = jnp.zeros((n_tok, 1, cout), jnp.float32)
    y = t1 + jnp.concatenate([z, t0[:, :-1]], axis=1) \
           + jnp.concatenate([t2[:, 1:], z], axis=1)
    return jnp.maximum(y + b_ref[e_idx][None], 0.0)


def _gate_route(emb3, gw, cb, g, be, w1t, b1, w2t, b2):
    b_total = emb3.shape[0]
    nb = b_total // _GB

    def body(emb_ref, gw_ref, cb_ref, g_ref, be_ref, w1_ref, b1_ref, w2_ref,
             b2_ref, out_ref):
        x2 = emb_ref[...].reshape(_GB * _L, _D)
        t0 = jnp.dot(x2, gw_ref[0, 0], preferred_element_type=jnp.float32)
        t1 = jnp.dot(x2, gw_ref[0, 1], preferred_element_type=jnp.float32)
        t2 = jnp.dot(x2, gw_ref[0, 2], preferred_element_type=jnp.float32)
        t0 = t0.reshape(_GB, _L, 64)
        t1 = t1.reshape(_GB, _L, 64)
        t2 = t2.reshape(_GB, _L, 64)
        z = jnp.zeros((_GB, 1, 64), jnp.float32)
        y = (jnp.concatenate([z, t0[:, :-1]], axis=1) + t1) \
            + jnp.concatenate([t2[:, 1:], z], axis=1)
        y = y + cb_ref[0][None]
        y = y / jnp.sqrt(jnp.float32(1.0 + _EPS)) * g_ref[0][None] \
            + be_ref[0][None]
        h = jnp.maximum(y, 0.0)
        hm = jnp.max(h, axis=1)
        h2 = jnp.maximum(
            jnp.dot(hm, w1_ref[...], preferred_element_type=jnp.float32)
            + b1_ref[...], 0.0)
        lg = jnp.dot(h2, w2_ref[...], preferred_element_type=jnp.float32) \
            + b2_ref[...]
        pm = jnp.exp(lg - jnp.max(lg, axis=1, keepdims=True))
        p = pm / jnp.sum(pm, axis=1, keepdims=True)
        mx = jnp.max(p, axis=1, keepdims=True)
        cand = jnp.where(p >= mx,
                         lax.broadcasted_iota(jnp.int32, p.shape, 1), _E)
        out_ref[0, 0] = jnp.min(cand, axis=1)

    out = pl.pallas_call(
        body,
        grid=(nb,),
        in_specs=[
            pl.BlockSpec((_GB, _L, _D), lambda i: (i, 0, 0)),
            pl.BlockSpec((1, 3, _D, 64), lambda i: (0, 0, 0, 0)),
            pl.BlockSpec((1, 1, 64), lambda i: (0, 0, 0)),
            pl.BlockSpec((1, 1, 64), lambda i: (0, 0, 0)),
            pl.BlockSpec((1, 1, 64), lambda i: (0, 0, 0)),
            pl.BlockSpec((64, 32), lambda i: (0, 0)),
            pl.BlockSpec((1, 32), lambda i: (0, 0)),
            pl.BlockSpec((32, _E), lambda i: (0, 0)),
            pl.BlockSpec((1, _E), lambda i: (0, 0)),
        ],
        out_specs=pl.BlockSpec((1, 1, _GB), lambda i: (i, 0, 0)),
        out_shape=jax.ShapeDtypeStruct((nb, 1, _GB), jnp.int32),
    )(emb3, gw, cb, g, be, w1t, b1, w2t, b2)
    return out.reshape(b_total)


def _expert_blocks(emb3, src, bexp, nused, w1, bb1, w2, bb2, w3, bb3,
                   f1w, f1b, f2w, f2b):
    p = src.shape[0]
    nblk = p // _CAP

    def body(src_ref, bexp_ref, nused_ref, emb_ref, w1_ref, b1_ref, w2_ref,
             b2_ref, w3_ref, b3_ref, f1w_ref, f1b_ref, f2w_ref, f2b_ref,
             out_ref, xbuf):
        g = pl.program_id(0)
        j = pl.program_id(1)
        xbuf[pl.ds(j, 1)] = emb_ref[...]

        @pl.when(jnp.logical_and(j == _CAP - 1, g < nused_ref[0]))
        def _():
            x2 = xbuf[...].reshape(_CAP * _L, _D)
            h1 = _shifted_conv(x2, _CAP, w1_ref, b1_ref, 0)
            h2 = _shifted_conv(h1.reshape(_CAP * _L, 64), _CAP, w2_ref,
                               b2_ref, 0)
            h3 = _shifted_conv(h2.reshape(_CAP * _L, 32), _CAP, w3_ref,
                               b3_ref, 0)
            m = jnp.max(h3, axis=1)
            f = jnp.maximum(
                jnp.dot(m, f1w_ref[...], preferred_element_type=jnp.float32)
                + f1b_ref[...], 0.0)
            o = jnp.dot(
                f, f2w_ref[...], preferred_element_type=jnp.float32) \
                + f2b_ref[...]
            out_ref[0] = jnp.concatenate(
                [o, jnp.zeros((_CAP, 112), jnp.float32)], axis=1)

    grid_spec = pltpu.PrefetchScalarGridSpec(
        num_scalar_prefetch=3,
        grid=(nblk, _CAP),
        in_specs=[
            pl.BlockSpec((1, _L, _D),
                         lambda g, j, src, bexp, nu: (src[g * _CAP + j], 0, 0)),
            pl.BlockSpec((1, 3, _D, 64),
                         lambda g, j, src, bexp, nu: (bexp[g], 0, 0, 0)),
            pl.BlockSpec((1, 1, 64),
                         lambda g, j, src, bexp, nu: (bexp[g], 0, 0)),
            pl.BlockSpec((1, 3, 64, 32),
                         lambda g, j, src, bexp, nu: (bexp[g], 0, 0, 0)),
            pl.BlockSpec((1, 1, 32),
                         lambda g, j, src, bexp, nu: (bexp[g], 0, 0)),
            pl.BlockSpec((1, 3, 32, 16),
                         lambda g, j, src, bexp, nu: (bexp[g], 0, 0, 0)),
            pl.BlockSpec((1, 1, 16),
                         lambda g, j, src, bexp, nu: (bexp[g], 0, 0)),
            pl.BlockSpec((16, 64), lambda g, j, src, bexp, nu: (0, 0)),
            pl.BlockSpec((1, 64), lambda g, j, src, bexp, nu: (0, 0)),
            pl.BlockSpec((64, 16), lambda g, j, src, bexp, nu: (0, 0)),
            pl.BlockSpec((1, 16), lambda g, j, src, bexp, nu: (0, 0)),
        ],
        out_specs=pl.BlockSpec((1, _CAP, 128),
                               lambda g, j, src, bexp, nu: (g, 0, 0)),
        scratch_shapes=[pltpu.VMEM((_CAP, _L, _D), jnp.float32)],
    )
    return pl.pallas_call(
        body,
        grid_spec=grid_spec,
        out_shape=jax.ShapeDtypeStruct((nblk, _CAP, 128), jnp.float32),
    )(src, bexp, nused, emb3, w1, bb1, w2, bb2, w3, bb3, f1w, f1b, f2w, f2b)


def _fold_conv(cw, cb, g, be):
    s = g / jnp.sqrt(1.0 + _EPS)
    w = jnp.transpose(cw * s[:, None, None], (2, 1, 0))
    b = (cb * s + be)[None, :]
    return w, b


def kernel(x, params):
    b_total = x.shape[0]

    emb_flat = _sc_gather(params["embedding"],
                          x.reshape(b_total * _L).astype(jnp.int32), 256)
    emb3 = emb_flat.reshape(b_total, _L, _D)

    gp = params["gate"]
    gw = jnp.transpose(gp["cw"], (2, 1, 0))
    top_idx = _gate_route(
        emb3, gw[None], gp["cb"][None, None, :], gp["g"][None, None, :],
        gp["be"][None, None, :], gp["w1"].T, gp["b1"][None, :],
        gp["w2"].T, gp["b2"][None, :])

    nblk = b_total // _CAP + _E
    p = nblk * _CAP
    oh = (top_idx[:, None] == jnp.arange(_E, dtype=jnp.int32)).astype(jnp.int32)
    rank = jnp.sum((jnp.cumsum(oh, axis=0) - oh) * oh, axis=1)
    counts = jnp.sum(oh, axis=0)
    blocks_e = (counts + _CAP - 1) // _CAP
    start_blk = jnp.concatenate(
        [jnp.zeros((1,), jnp.int32), jnp.cumsum(blocks_e)[:-1]])
    dest = start_blk[top_idx] * _CAP + rank
    src = jnp.zeros((p,), jnp.int32).at[dest].set(
        jnp.arange(b_total, dtype=jnp.int32))
    gids = jnp.arange(nblk, dtype=jnp.int32)
    bexp = jnp.sum(gids[:, None] >= start_blk[None, :], axis=1,
                   dtype=jnp.int32) - 1
    nused = jnp.sum(blocks_e, dtype=jnp.int32)[None]

    ws1, ws2, ws3, bs1, bs2, bs3 = [], [], [], [], [], []
    for ep in params["experts"]:
        w1, b1 = _fold_conv(ep["w1"], ep["b1"], ep["g1"], ep["be1"])
        w2, b2 = _fold_conv(ep["w2"], ep["b2"], ep["g2"], ep["be2"])
        w3, b3 = _fold_conv(ep["w3"], ep["b3"], ep["g3"], ep["be3"])
        ws1.append(w1); ws2.append(w2); ws3.append(w3)
        bs1.append(b1); bs2.append(b2); bs3.append(b3)
    f2w = jnp.zeros((64, 16), jnp.float32).at[:, :2].set(params["fc2_w"].T)
    f2b = jnp.zeros((1, 16), jnp.float32).at[:, :2].set(params["fc2_b"][None])
    out_p = _expert_blocks(
        emb3, src, bexp, nused,
        jnp.stack(ws1), jnp.stack(bs1), jnp.stack(ws2), jnp.stack(bs2),
        jnp.stack(ws3), jnp.stack(bs3),
        params["fc1_w"].T, params["fc1_b"][None, :], f2w, f2b)

    out_rows = _sc_gather(out_p.reshape(p, 128), dest, 128)
    return out_rows[:, :2]


# speedup vs baseline: 1.9266x; 1.9266x over previous
import functools

import jax
import jax.numpy as jnp
from jax import lax
from jax.experimental import pallas as pl
from jax.experimental.pallas import tpu as pltpu
from jax.experimental.pallas import tpu_sc as plsc

_D = 128
_L = 50
_E = 8
_EPS = 1e-5
_CAP = 128
_GB = 128

_NC, _NS = 2, 16
_NW = _NC * _NS


def _sc_gather(table, idx, chunk):
    m, d = idx.shape[0], table.shape[1]
    per_w = m // _NW
    n_chunks = per_w // chunk
    mesh = plsc.VectorSubcoreMesh(
        core_axis_name="c", subcore_axis_name="s",
        num_cores=_NC, num_subcores=_NS)

    @functools.partial(
        pl.kernel, mesh=mesh,
        out_type=jax.ShapeDtypeStruct((m, d), table.dtype),
        scratch_types=[
            pltpu.VMEM((chunk,), jnp.int32),
            pltpu.VMEM((chunk, d), table.dtype),
            pltpu.SemaphoreType.DMA,
        ],
    )
    def k(table_hbm, idx_hbm, out_hbm, idx_v, rows_v, sem):
        wid = lax.axis_index("s") * _NC + lax.axis_index("c")
        base = wid * per_w

        def body(c, carry):
            off = base + c * chunk
            pltpu.sync_copy(idx_hbm.at[pl.ds(off, chunk)], idx_v)
            pltpu.async_copy(table_hbm.at[idx_v], rows_v, sem).wait()
            pltpu.sync_copy(rows_v, out_hbm.at[pl.ds(off, chunk)])
            return carry

        lax.fori_loop(0, n_chunks, body, 0)

    return k(table, idx)


def _shifted_conv(x_flat, n_tok, w_ref, b_ref, e_idx):
    cout = w_ref.shape[3]
    t0 = jnp.dot(x_flat, w_ref[e_idx, 0], preferred_element_type=jnp.float32)
    t1 = jnp.dot(x_flat, w_ref[e_idx, 1], preferred_element_type=jnp.float32)
    t2 = jnp.dot(x_flat, w_ref[e_idx, 2], preferred_element_type=jnp.float32)
    t0 = t0.reshape(n_tok, _L, cout)
    t1 = t1.reshape(n_tok, _L, cout)
    t2 = t2.reshape(n_tok, _L, cout)
    z = jnp.zeros((n_tok, 1, cout), jnp.float32)
    y = t1 + jnp.concatenate([z, t0[:, :-1]], axis=1) \
           + jnp.concatenate([t2[:, 1:], z], axis=1)
    return jnp.maximum(y + b_ref[e_idx][None], 0.0)


def _gate_route(emb3, gw, cb, g, be, w1t, b1, w2t, b2):
    b_total = emb3.shape[0]
    nb = b_total // _GB

    def body(emb_ref, gw_ref, cb_ref, g_ref, be_ref, w1_ref, b1_ref, w2_ref,
             b2_ref, out_ref):
        x2 = emb_ref[...].reshape(_GB * _L, _D)
        t0 = jnp.dot(x2, gw_ref[0, 0], preferred_element_type=jnp.float32)
        t1 = jnp.dot(x2, gw_ref[0, 1], preferred_element_type=jnp.float32)
        t2 = jnp.dot(x2, gw_ref[0, 2], preferred_element_type=jnp.float32)
        t0 = t0.reshape(_GB, _L, 64)
        t1 = t1.reshape(_GB, _L, 64)
        t2 = t2.reshape(_GB, _L, 64)
        z = jnp.zeros((_GB, 1, 64), jnp.float32)
        y = (jnp.concatenate([z, t0[:, :-1]], axis=1) + t1) \
            + jnp.concatenate([t2[:, 1:], z], axis=1)
        y = y + cb_ref[0][None]
        y = y / jnp.sqrt(jnp.float32(1.0 + _EPS)) * g_ref[0][None] \
            + be_ref[0][None]
        h = jnp.maximum(y, 0.0)
        hm = jnp.max(h, axis=1)
        h2 = jnp.maximum(
            jnp.dot(hm, w1_ref[...], preferred_element_type=jnp.float32)
            + b1_ref[...], 0.0)
        lg = jnp.dot(h2, w2_ref[...], preferred_element_type=jnp.float32) \
            + b2_ref[...]
        pm = jnp.exp(lg - jnp.max(lg, axis=1, keepdims=True))
        p = pm / jnp.sum(pm, axis=1, keepdims=True)
        mx = jnp.max(p, axis=1, keepdims=True)
        cand = jnp.where(p >= mx,
                         lax.broadcasted_iota(jnp.int32, p.shape, 1), _E)
        out_ref[0, 0] = jnp.min(cand, axis=1)

    out = pl.pallas_call(
        body,
        grid=(nb,),
        in_specs=[
            pl.BlockSpec((_GB, _L, _D), lambda i: (i, 0, 0)),
            pl.BlockSpec((1, 3, _D, 64), lambda i: (0, 0, 0, 0)),
            pl.BlockSpec((1, 1, 64), lambda i: (0, 0, 0)),
            pl.BlockSpec((1, 1, 64), lambda i: (0, 0, 0)),
            pl.BlockSpec((1, 1, 64), lambda i: (0, 0, 0)),
            pl.BlockSpec((64, 32), lambda i: (0, 0)),
            pl.BlockSpec((1, 32), lambda i: (0, 0)),
            pl.BlockSpec((32, _E), lambda i: (0, 0)),
            pl.BlockSpec((1, _E), lambda i: (0, 0)),
        ],
        out_specs=pl.BlockSpec((1, 1, _GB), lambda i: (i, 0, 0)),
        out_shape=jax.ShapeDtypeStruct((nb, 1, _GB), jnp.int32),
    )(emb3, gw, cb, g, be, w1t, b1, w2t, b2)
    return out.reshape(b_total)


def _expert_blocks(emb_p, bexp, nused, w1, bb1, w2, bb2, w3, bb3,
                   f1w, f1b, f2w, f2b):
    nblk = emb_p.shape[0] // _CAP

    def body(bexp_ref, nused_ref, emb_ref, w1_ref, b1_ref, w2_ref,
             b2_ref, w3_ref, b3_ref, f1w_ref, f1b_ref, f2w_ref, f2b_ref,
             out_ref):
        g = pl.program_id(0)

        @pl.when(g < nused_ref[0])
        def _():
            x2 = emb_ref[...].reshape(_CAP * _L, _D)
            h1 = _shifted_conv(x2, _CAP, w1_ref, b1_ref, 0)
            h2 = _shifted_conv(h1.reshape(_CAP * _L, 64), _CAP, w2_ref,
                               b2_ref, 0)
            h3 = _shifted_conv(h2.reshape(_CAP * _L, 32), _CAP, w3_ref,
                               b3_ref, 0)
            m = jnp.max(h3, axis=1)
            f = jnp.maximum(
                jnp.dot(m, f1w_ref[...], preferred_element_type=jnp.float32)
                + f1b_ref[...], 0.0)
            o = jnp.dot(
                f, f2w_ref[...], preferred_element_type=jnp.float32) \
                + f2b_ref[...]
            out_ref[0] = jnp.concatenate(
                [o, jnp.zeros((_CAP, 112), jnp.float32)], axis=1)

    grid_spec = pltpu.PrefetchScalarGridSpec(
        num_scalar_prefetch=2,
        grid=(nblk,),
        in_specs=[
            pl.BlockSpec((_CAP, _L, _D),
                         lambda g, bexp, nu: (g, 0, 0)),
            pl.BlockSpec((1, 3, _D, 64),
                         lambda g, bexp, nu: (bexp[g], 0, 0, 0)),
            pl.BlockSpec((1, 1, 64),
                         lambda g, bexp, nu: (bexp[g], 0, 0)),
            pl.BlockSpec((1, 3, 64, 32),
                         lambda g, bexp, nu: (bexp[g], 0, 0, 0)),
            pl.BlockSpec((1, 1, 32),
                         lambda g, bexp, nu: (bexp[g], 0, 0)),
            pl.BlockSpec((1, 3, 32, 16),
                         lambda g, bexp, nu: (bexp[g], 0, 0, 0)),
            pl.BlockSpec((1, 1, 16),
                         lambda g, bexp, nu: (bexp[g], 0, 0)),
            pl.BlockSpec((16, 64), lambda g, bexp, nu: (0, 0)),
            pl.BlockSpec((1, 64), lambda g, bexp, nu: (0, 0)),
            pl.BlockSpec((64, 16), lambda g, bexp, nu: (0, 0)),
            pl.BlockSpec((1, 16), lambda g, bexp, nu: (0, 0)),
        ],
        out_specs=pl.BlockSpec((1, _CAP, 128),
                               lambda g, bexp, nu: (g, 0, 0)),
    )
    return pl.pallas_call(
        body,
        grid_spec=grid_spec,
        out_shape=jax.ShapeDtypeStruct((nblk, _CAP, 128), jnp.float32),
    )(bexp, nused, emb_p, w1, bb1, w2, bb2, w3, bb3, f1w, f1b, f2w, f2b)


def _fold_conv(cw, cb, g, be):
    s = g / jnp.sqrt(1.0 + _EPS)
    w = jnp.transpose(cw * s[:, None, None], (2, 1, 0))
    b = (cb * s + be)[None, :]
    return w, b


def kernel(x, params):
    b_total = x.shape[0]

    emb_flat = _sc_gather(params["embedding"],
                          x.reshape(b_total * _L).astype(jnp.int32), 256)
    emb3 = emb_flat.reshape(b_total, _L, _D)

    gp = params["gate"]
    gw = jnp.transpose(gp["cw"], (2, 1, 0))
    top_idx = _gate_route(
        emb3, gw[None], gp["cb"][None, None, :], gp["g"][None, None, :],
        gp["be"][None, None, :], gp["w1"].T, gp["b1"][None, :],
        gp["w2"].T, gp["b2"][None, :])

    nblk = b_total // _CAP + _E
    p = nblk * _CAP
    oh = (top_idx[:, None] == jnp.arange(_E, dtype=jnp.int32)).astype(jnp.int32)
    rank = jnp.sum((jnp.cumsum(oh, axis=0) - oh) * oh, axis=1)
    counts = jnp.sum(oh, axis=0)
    blocks_e = (counts + _CAP - 1) // _CAP
    start_blk = jnp.concatenate(
        [jnp.zeros((1,), jnp.int32), jnp.cumsum(blocks_e)[:-1]])
    dest = start_blk[top_idx] * _CAP + rank
    src = jnp.zeros((p,), jnp.int32).at[dest].set(
        jnp.arange(b_total, dtype=jnp.int32))
    gids = jnp.arange(nblk, dtype=jnp.int32)
    bexp = jnp.sum(gids[:, None] >= start_blk[None, :], axis=1,
                   dtype=jnp.int32) - 1
    nused = jnp.sum(blocks_e, dtype=jnp.int32)[None]

    ws1, ws2, ws3, bs1, bs2, bs3 = [], [], [], [], [], []
    for ep in params["experts"]:
        w1, b1 = _fold_conv(ep["w1"], ep["b1"], ep["g1"], ep["be1"])
        w2, b2 = _fold_conv(ep["w2"], ep["b2"], ep["g2"], ep["be2"])
        w3, b3 = _fold_conv(ep["w3"], ep["b3"], ep["g3"], ep["be3"])
        ws1.append(w1); ws2.append(w2); ws3.append(w3)
        bs1.append(b1); bs2.append(b2); bs3.append(b3)
    f2w = jnp.zeros((64, 16), jnp.float32).at[:, :2].set(params["fc2_w"].T)
    f2b = jnp.zeros((1, 16), jnp.float32).at[:, :2].set(params["fc2_b"][None])
    xp_idx = (src[:, None] * _L
              + jnp.arange(_L, dtype=jnp.int32)[None, :]).reshape(p * _L)
    emb_p = _sc_gather(emb_flat, xp_idx, 200).reshape(p, _L, _D)
    out_p = _expert_blocks(
        emb_p, bexp, nused,
        jnp.stack(ws1), jnp.stack(bs1), jnp.stack(ws2), jnp.stack(bs2),
        jnp.stack(ws3), jnp.stack(bs3),
        params["fc1_w"].T, params["fc1_b"][None, :], f2w, f2b)

    out_rows = _sc_gather(out_p.reshape(p, 128), dest, 128)
    return out_rows[:, :2]
